# Initial kernel scaffold; baseline (speedup 1.0000x reference)
import jax
import jax.numpy as jnp
from jax.experimental import pallas as pl


def kernel(xq, xc, edge_index_q, edge_index_c, qgraph_sizes, cgraph_sizes, Wg0, bg0, Wg1, bg1, Wg2, bg2, W_att, W_ntn_a, W_ntn_b, ntn_bias, W_fc1, b_fc1, W_fc2, b_fc2):
    raise NotImplementedError("write your pallas kernel here")


# speedup vs baseline: 54.5775x; 54.5775x over previous
import functools

import jax
import jax.numpy as jnp
from jax import lax
from jax.experimental import pallas as pl
from jax.experimental.pallas import tpu as pltpu
from jax.experimental.pallas import tpu_sc as plsc

B = 128
N = 80
D = 128
F = 128
EG = 1280
T = 16
BOT = 16

NN = N * N
NUM_ITEMS = 2 * B
NUM_WORKERS = 32
PER_W = NUM_ITEMS // NUM_WORKERS

GB = 8


def _build_counts(src, dst):
    mesh = plsc.VectorSubcoreMesh(
        core_axis_name="c", subcore_axis_name="s", num_cores=2, num_subcores=16
    )

    @functools.partial(
        pl.kernel,
        out_type=jax.ShapeDtypeStruct((NUM_ITEMS, NN), jnp.float32),
        mesh=mesh,
        scratch_types=[
            pltpu.VMEM((EG,), jnp.int32),
            pltpu.VMEM((EG,), jnp.int32),
            pltpu.VMEM((NN,), jnp.float32),
        ],
        compiler_params=pltpu.CompilerParams(needs_layout_passes=False),
    )
    def sc_kernel(src_hbm, dst_hbm, out_hbm, s_v, d_v, acc_v):
        wid = lax.axis_index("s") * 2 + lax.axis_index("c")
        zeros16 = jnp.zeros((16,), jnp.float32)
        ones16 = jnp.ones((16,), jnp.float32)

        @pl.loop(0, PER_W)
        def _(t):
            item = wid * PER_W + t
            bb = lax.rem(item, B) * (NN + N)
            pltpu.sync_copy(src_hbm.at[item], s_v)
            pltpu.sync_copy(dst_hbm.at[item], d_v)

            @pl.loop(0, NN, step=16)
            def _(i):
                acc_v[pl.ds(i, 16)] = zeros16

            @pl.loop(0, EG, step=16)
            def _(e):
                s = s_v[pl.ds(e, 16)]
                d = d_v[pl.ds(e, 16)]
                off = d * N + s - bb
                plsc.addupdate_scatter(acc_v, [off], ones16)

            pltpu.sync_copy(acc_v, out_hbm.at[item])

    return sc_kernel(src, dst)


def _gcn_pool_side(x3, c3, szb, wg, bg, watt_t):
    h = x3.reshape(GB * N, D)
    row = lax.broadcasted_iota(jnp.int32, (N, N), 0)
    col = lax.broadcasted_iota(jnp.int32, (N, N), 1)
    eye = jnp.where(row == col, 1.0, 0.0).astype(jnp.float32)

    adjs = []
    dinvs = []
    for g in range(GB):
        cg = c3[g] + eye
        deg = jnp.sum(cg, axis=1, keepdims=True)
        dinvs.append(lax.rsqrt(deg))
        adjs.append(cg)

    for w, b in zip(wg, bg):
        hw = jnp.dot(h, w, preferred_element_type=jnp.float32, precision=lax.Precision.HIGHEST)
        outs = []
        for g in range(GB):
            hwg = hw[g * N:(g + 1) * N]
            u = jnp.dot(adjs[g], dinvs[g] * hwg,
                        preferred_element_type=jnp.float32, precision=lax.Precision.HIGHEST)
            outs.append(dinvs[g] * u)
        h = jax.nn.relu(jnp.concatenate(outs, axis=0) + b)

    es = []
    for g in range(GB):
        qg = h[g * N:(g + 1) * N]
        size = szb[g:g + 1, :]
        csum = jnp.sum(qg, axis=0, keepdims=True) / size
        ctx = jnp.tanh(jnp.dot(csum, watt_t,
                               preferred_element_type=jnp.float32, precision=lax.Precision.HIGHEST))
        sg = jax.nn.sigmoid(jnp.sum(qg * ctx, axis=1, keepdims=True))
        es.append(jnp.sum(qg * sg, axis=0, keepdims=True))
    return jnp.concatenate(es, axis=0)


def _stage1_body(xq_ref, xc_ref, cq_ref, cc_ref, szq_ref, szc_ref,
                 wg0_ref, bg0_ref, wg1_ref, bg1_ref, wg2_ref, bg2_ref,
                 watt_ref, e1_ref, e2_ref):
    wg = [wg0_ref[...], wg1_ref[...], wg2_ref[...]]
    bg = [bg0_ref[...], bg1_ref[...], bg2_ref[...]]
    watt_t = watt_ref[...]
    e1_ref[...] = _gcn_pool_side(xq_ref[...], cq_ref[...], szq_ref[...],
                                 wg, bg, watt_t)
    e2_ref[...] = _gcn_pool_side(xc_ref[...], cc_ref[...], szc_ref[...],
                                 wg, bg, watt_t)


def _stage2_body(e1_ref, e2_ref, war_ref, wb1_ref, wb2_ref, nb_ref,
                 wfc1_ref, bfc1_ref, wfc2_ref, bfc2_ref, out_ref):
    e1 = e1_ref[...]
    e2 = e2_ref[...]
    u = jnp.dot(e1, war_ref[...], preferred_element_type=jnp.float32, precision=lax.Precision.HIGHEST)
    u3 = u.reshape(B, T, F)
    bil = jnp.sum(u3 * e2[:, None, :], axis=2)
    scores = jax.nn.relu(
        bil
        + jnp.dot(e1, wb1_ref[...], preferred_element_type=jnp.float32, precision=lax.Precision.HIGHEST)
        + jnp.dot(e2, wb2_ref[...], preferred_element_type=jnp.float32, precision=lax.Precision.HIGHEST)
        + nb_ref[...])
    h = jax.nn.relu(jnp.dot(scores, wfc1_ref[...],
                            preferred_element_type=jnp.float32, precision=lax.Precision.HIGHEST) + bfc1_ref[...])
    sc = jax.nn.sigmoid(jnp.dot(h, wfc2_ref[...],
                                preferred_element_type=jnp.float32, precision=lax.Precision.HIGHEST)
                        + bfc2_ref[...][:, :1])
    out_ref[...] = jnp.broadcast_to(sc, (B, 128))


def _stage1(xq3, xc3, cq3, cc3, szqb, szcb, wg0, bg0, wg1, bg1, wg2, bg2,
            watt_t):
    full = lambda i: (0, 0)
    blk = lambda i: (i, 0)
    blk3 = lambda i: (i, 0, 0)
    return pl.pallas_call(
        _stage1_body,
        grid=(B // GB,),
        in_specs=[
            pl.BlockSpec((GB, N, D), blk3),
            pl.BlockSpec((GB, N, D), blk3),
            pl.BlockSpec((GB, N, N), blk3),
            pl.BlockSpec((GB, N, N), blk3),
            pl.BlockSpec((GB, 128), blk),
            pl.BlockSpec((GB, 128), blk),
            pl.BlockSpec((D, F), full),
            pl.BlockSpec((1, F), full),
            pl.BlockSpec((F, F), full),
            pl.BlockSpec((1, F), full),
            pl.BlockSpec((F, F), full),
            pl.BlockSpec((1, F), full),
            pl.BlockSpec((F, F), full),
        ],
        out_specs=[
            pl.BlockSpec((GB, F), blk),
            pl.BlockSpec((GB, F), blk),
        ],
        out_shape=[
            jax.ShapeDtypeStruct((B, F), jnp.float32),
            jax.ShapeDtypeStruct((B, F), jnp.float32),
        ],
    )(xq3, xc3, cq3, cc3, szqb, szcb, wg0, bg0, wg1, bg1, wg2, bg2, watt_t)


def _stage2(e1, e2, war, wb1, wb2, nb, wfc1, bfc1, wfc2, bfc2):
    full = lambda: (0, 0)
    return pl.pallas_call(
        _stage2_body,
        out_shape=jax.ShapeDtypeStruct((B, 128), jnp.float32),
    )(e1, e2, war, wb1, wb2, nb, wfc1, bfc1, wfc2, bfc2)


def kernel(xq, xc, edge_index_q, edge_index_c, qgraph_sizes, cgraph_sizes,
           Wg0, bg0, Wg1, bg1, Wg2, bg2, W_att, W_ntn_a, W_ntn_b, ntn_bias,
           W_fc1, b_fc1, W_fc2, b_fc2):
    eq = edge_index_q.astype(jnp.int32)
    ec = edge_index_c.astype(jnp.int32)
    src = jnp.concatenate([eq[0].reshape(B, EG), ec[0].reshape(B, EG)], axis=0)
    dst = jnp.concatenate([eq[1].reshape(B, EG), ec[1].reshape(B, EG)], axis=0)

    counts = _build_counts(src, dst)
    cq3 = counts[:B].reshape(B, N, N)
    cc3 = counts[B:].reshape(B, N, N)

    xq3 = xq.reshape(B, N, D)
    xc3 = xc.reshape(B, N, D)
    szqb = jnp.broadcast_to(qgraph_sizes[:, None], (B, 128))
    szcb = jnp.broadcast_to(cgraph_sizes[:, None], (B, 128))

    e1, e2 = _stage1(xq3, xc3, cq3, cc3, szqb, szcb,
                     Wg0, bg0.reshape(1, F), Wg1, bg1.reshape(1, F),
                     Wg2, bg2.reshape(1, F), W_att.T)

    war = W_ntn_a.transpose(1, 0, 2).reshape(F, T * F)
    wb1 = W_ntn_b[:, :F].T
    wb2 = W_ntn_b[:, F:].T
    nb = ntn_bias.reshape(1, T)
    wfc1 = W_fc1.T
    bfc1 = b_fc1.reshape(1, BOT)
    wfc2 = W_fc2.T
    bfc2 = jnp.broadcast_to(b_fc2.reshape(1, 1), (1, 128))

    out = _stage2(e1, e2, war, wb1, wb2, nb, wfc1, bfc1, wfc2, bfc2)
    return out[:, 0]
